# baseline (speedup 1.0000x reference)
import functools

import jax
import jax.numpy as jnp
from jax import lax
from jax.experimental import pallas as pl
from jax.experimental.pallas import tpu as pltpu
from jax.experimental.pallas import tpu_sc as plsc

_EMB = 32
_B = 16384
_V = 1000000
_TB = 32768
_QB = _TB // 4
_GRID = -(-_V // _TB)
_VP = _GRID * _TB
_NC = 2
_NS = 16
_NW = _NC * _NS
_BPW = _B // _NW
_NBLK = _BPW // 16
_CHUNK = 128
_NCHUNK = _BPW // _CHUNK
_SLOT = 32


def _tc_transpose_body(tt_ref, out_ref):
    y = jnp.concatenate(
        [tt_ref[:, pl.ds(_QB * q, _QB)] for q in range(4)], axis=0)
    out_ref[...] = jnp.transpose(y)


@jax.jit
def _tc_transpose(tt):
    return pl.pallas_call(
        _tc_transpose_body,
        grid=(_GRID,),
        in_specs=[pl.BlockSpec((_EMB, _TB), lambda g: (0, g))],
        out_specs=pl.BlockSpec((_QB, 128), lambda g: (g, 0)),
        out_shape=jax.ShapeDtypeStruct((_VP // 4, 128), jnp.float32),
    )(tt)


def _sc_body(idx1_hbm, idx2_hbm, sd_hbm, table_hbm, wb_hbm, out_hbm,
             idx1_v, idx2_v, sd_v, rows1_v, rows2_v, w_v, red_v, o_v, sem):
    wid = lax.axis_index("s") * _NC + lax.axis_index("c")
    base = wid * _BPW
    lane = lax.iota(jnp.int32, 16)
    zeros = jnp.zeros((16,), jnp.float32)

    pltpu.sync_copy(idx1_hbm.at[pl.ds(base, _BPW)], idx1_v)
    pltpu.sync_copy(idx2_hbm.at[pl.ds(base, _BPW)], idx2_v)
    pltpu.sync_copy(sd_hbm.at[pl.ds(base, _BPW)], sd_v)
    pltpu.sync_copy(wb_hbm, w_v)

    for iv in (idx1_v, idx2_v):
        for i in range(_NBLK):
            sl = pl.ds(i * 16, 16)
            t = iv[sl]
            rem = jnp.bitwise_and(t, _TB - 1)
            q = jnp.right_shift(rem, _QB.bit_length() - 1)
            j = jnp.bitwise_and(rem, _QB - 1)
            iv[sl] = (t - rem) + jnp.left_shift(j, 2) + q

    copies = []
    for j in range(_NCHUNK):
        rsl = pl.ds(j * _CHUNK, _CHUNK)
        copies.append(pltpu.async_copy(
            table_hbm.at[idx1_v.at[rsl]], rows1_v.at[rsl], sem))
        copies.append(pltpu.async_copy(
            table_hbm.at[idx2_v.at[rsl]], rows2_v.at[rsl], sem))

    for j in range(16 * _SLOT // 16):
        red_v[pl.ds(16 * j, 16)] = zeros

    w1lo = w_v[pl.ds(0, 16)]
    w1hi = w_v[pl.ds(16, 16)]
    w2lo = w_v[pl.ds(32, 16)]
    w2hi = w_v[pl.ds(48, 16)]
    wtail = w_v[pl.ds(64, 16)]
    w_sd = wtail[0]
    b0 = wtail[1]
    masks = [lane == r for r in range(16)]

    for cp in copies:
        cp.wait()

    def block(i, _):
        sl = pl.ds(i * 16, 16)
        acc = sd_v[sl] * w_sd + b0
        for rr in range(16):
            r = i * 16 + rr
            v = (rows1_v[r, pl.ds(0, 16)] * w1lo
                 + rows1_v[r, pl.ds(16, 16)] * w1hi
                 + rows2_v[r, pl.ds(0, 16)] * w2lo
                 + rows2_v[r, pl.ds(16, 16)] * w2hi)
            s = rr * _SLOT
            red_v[pl.ds(s, 16)] = v
            v = v + red_v[pl.ds(s + 8, 16)]
            red_v[pl.ds(s, 16)] = v
            v = v + red_v[pl.ds(s + 4, 16)]
            red_v[pl.ds(s, 16)] = v
            v = v + red_v[pl.ds(s + 2, 16)]
            red_v[pl.ds(s, 16)] = v
            v = v + red_v[pl.ds(s + 1, 16)]
            acc = acc + jnp.where(masks[rr], v[0], 0.0)
        o_v[sl] = 1.0 / (1.0 + jnp.exp(-acc))
        return _

    lax.fori_loop(0, _NBLK, block, None)

    pltpu.sync_copy(o_v, out_hbm.at[pl.ds(base, _BPW)])


@jax.jit
def _sc_forward(idx1, idx2, sd, table_lin, wb):
    mesh = plsc.VectorSubcoreMesh(core_axis_name="c", subcore_axis_name="s")
    f = functools.partial(
        pl.kernel,
        mesh=mesh,
        compiler_params=pltpu.CompilerParams(use_tc_tiling_on_sc=False),
        out_type=jax.ShapeDtypeStruct((_B,), jnp.float32),
        scratch_types=[
            pltpu.VMEM((_BPW,), jnp.int32),
            pltpu.VMEM((_BPW,), jnp.int32),
            pltpu.VMEM((_BPW,), jnp.float32),
            pltpu.VMEM((_BPW, _EMB), jnp.float32),
            pltpu.VMEM((_BPW, _EMB), jnp.float32),
            pltpu.VMEM((80,), jnp.float32),
            pltpu.VMEM((16 * _SLOT,), jnp.float32),
            pltpu.VMEM((_BPW,), jnp.float32),
            pltpu.SemaphoreType.DMA,
        ],
    )(_sc_body)
    return f(idx1, idx2, sd, table_lin, wb)


def kernel(idsTensor, table, W, b):
    idx1 = idsTensor[:, 0].astype(jnp.int32)
    idx2 = idsTensor[:, 1].astype(jnp.int32)
    sd = idsTensor[:, 2].astype(jnp.bfloat16).astype(jnp.float32)
    wbf = W.reshape(-1).astype(jnp.bfloat16).astype(jnp.float32)
    wb = jnp.concatenate(
        [wbf, b, jnp.zeros((14,), jnp.float32)])
    table_lin = _tc_transpose(table.T).reshape(_VP, _EMB)
    out = _sc_forward(idx1, idx2, sd, table_lin, wb)
    return out.reshape(_B, 1)


# speedup vs baseline: 1.0106x; 1.0106x over previous
import functools

import jax
import jax.numpy as jnp
from jax import lax
from jax.experimental import pallas as pl
from jax.experimental.pallas import tpu as pltpu
from jax.experimental.pallas import tpu_sc as plsc

_EMB = 32
_B = 16384
_V = 1000000
_TB = 65536
_QB = _TB // 4
_GRID = -(-_V // _TB)
_VP = _GRID * _TB
_NC = 2
_NS = 16
_NW = _NC * _NS
_BPW = _B // _NW
_NBLK = _BPW // 16
_CHUNK = 128
_NCHUNK = _BPW // _CHUNK
_SLOT = 32


def _tc_transpose_body(tt_ref, out_ref):
    y = jnp.concatenate(
        [tt_ref[:, pl.ds(_QB * q, _QB)] for q in range(4)], axis=0)
    out_ref[...] = jnp.transpose(y)


@jax.jit
def _tc_transpose(tt):
    return pl.pallas_call(
        _tc_transpose_body,
        grid=(_GRID,),
        in_specs=[pl.BlockSpec((_EMB, _TB), lambda g: (0, g))],
        out_specs=pl.BlockSpec((_QB, 128), lambda g: (g, 0)),
        out_shape=jax.ShapeDtypeStruct((_VP // 4, 128), jnp.float32),
    )(tt)


def _sc_body(idx1_hbm, idx2_hbm, sd_hbm, table_hbm, wb_hbm, out_hbm,
             idx1_v, idx2_v, sd_v, rows1_v, rows2_v, w_v, red_v, o_v, sem):
    wid = lax.axis_index("s") * _NC + lax.axis_index("c")
    base = wid * _BPW
    lane = lax.iota(jnp.int32, 16)
    zeros = jnp.zeros((16,), jnp.float32)

    pltpu.sync_copy(idx1_hbm.at[pl.ds(base, _BPW)], idx1_v)
    pltpu.sync_copy(idx2_hbm.at[pl.ds(base, _BPW)], idx2_v)
    pltpu.sync_copy(sd_hbm.at[pl.ds(base, _BPW)], sd_v)
    pltpu.sync_copy(wb_hbm, w_v)

    for iv in (idx1_v, idx2_v):
        for i in range(_NBLK):
            sl = pl.ds(i * 16, 16)
            t = iv[sl]
            rem = jnp.bitwise_and(t, _TB - 1)
            q = jnp.right_shift(rem, _QB.bit_length() - 1)
            j = jnp.bitwise_and(rem, _QB - 1)
            iv[sl] = (t - rem) + jnp.left_shift(j, 2) + q

    copies = []
    for j in range(_NCHUNK):
        rsl = pl.ds(j * _CHUNK, _CHUNK)
        copies.append(pltpu.async_copy(
            table_hbm.at[idx1_v.at[rsl]], rows1_v.at[rsl], sem))
        copies.append(pltpu.async_copy(
            table_hbm.at[idx2_v.at[rsl]], rows2_v.at[rsl], sem))

    for j in range(16 * _SLOT // 16):
        red_v[pl.ds(16 * j, 16)] = zeros

    w1lo = w_v[pl.ds(0, 16)]
    w1hi = w_v[pl.ds(16, 16)]
    w2lo = w_v[pl.ds(32, 16)]
    w2hi = w_v[pl.ds(48, 16)]
    wtail = w_v[pl.ds(64, 16)]
    w_sd = wtail[0]
    b0 = wtail[1]
    masks = [lane == r for r in range(16)]

    for cp in copies:
        cp.wait()

    def block(i, _):
        sl = pl.ds(i * 16, 16)
        acc = sd_v[sl] * w_sd + b0
        for rr in range(16):
            r = i * 16 + rr
            v = (rows1_v[r, pl.ds(0, 16)] * w1lo
                 + rows1_v[r, pl.ds(16, 16)] * w1hi
                 + rows2_v[r, pl.ds(0, 16)] * w2lo
                 + rows2_v[r, pl.ds(16, 16)] * w2hi)
            s = rr * _SLOT
            red_v[pl.ds(s, 16)] = v
            v = v + red_v[pl.ds(s + 8, 16)]
            red_v[pl.ds(s, 16)] = v
            v = v + red_v[pl.ds(s + 4, 16)]
            red_v[pl.ds(s, 16)] = v
            v = v + red_v[pl.ds(s + 2, 16)]
            red_v[pl.ds(s, 16)] = v
            v = v + red_v[pl.ds(s + 1, 16)]
            acc = acc + jnp.where(masks[rr], v[0], 0.0)
        o_v[sl] = 1.0 / (1.0 + jnp.exp(-acc))
        return _

    lax.fori_loop(0, _NBLK, block, None)

    pltpu.sync_copy(o_v, out_hbm.at[pl.ds(base, _BPW)])


@jax.jit
def _sc_forward(idx1, idx2, sd, table_lin, wb):
    mesh = plsc.VectorSubcoreMesh(core_axis_name="c", subcore_axis_name="s")
    f = functools.partial(
        pl.kernel,
        mesh=mesh,
        compiler_params=pltpu.CompilerParams(use_tc_tiling_on_sc=False),
        out_type=jax.ShapeDtypeStruct((_B,), jnp.float32),
        scratch_types=[
            pltpu.VMEM((_BPW,), jnp.int32),
            pltpu.VMEM((_BPW,), jnp.int32),
            pltpu.VMEM((_BPW,), jnp.float32),
            pltpu.VMEM((_BPW, _EMB), jnp.float32),
            pltpu.VMEM((_BPW, _EMB), jnp.float32),
            pltpu.VMEM((80,), jnp.float32),
            pltpu.VMEM((16 * _SLOT,), jnp.float32),
            pltpu.VMEM((_BPW,), jnp.float32),
            pltpu.SemaphoreType.DMA,
        ],
    )(_sc_body)
    return f(idx1, idx2, sd, table_lin, wb)


def kernel(idsTensor, table, W, b):
    idx1 = idsTensor[:, 0].astype(jnp.int32)
    idx2 = idsTensor[:, 1].astype(jnp.int32)
    sd = idsTensor[:, 2].astype(jnp.bfloat16).astype(jnp.float32)
    wbf = W.reshape(-1).astype(jnp.bfloat16).astype(jnp.float32)
    wb = jnp.concatenate(
        [wbf, b, jnp.zeros((14,), jnp.float32)])
    table_lin = _tc_transpose(table.T).reshape(_VP, _EMB)
    out = _sc_forward(idx1, idx2, sd, table_lin, wb)
    return out.reshape(_B, 1)


# speedup vs baseline: 1.0263x; 1.0155x over previous
import functools

import jax
import jax.numpy as jnp
from jax import lax
from jax.experimental import pallas as pl
from jax.experimental.pallas import tpu as pltpu
from jax.experimental.pallas import tpu_sc as plsc

_EMB = 32
_B = 16384
_V = 1000000
_TB = 65536
_QB = _TB // 4
_GRID = -(-_V // _TB)
_VP = _GRID * _TB
_NC = 2
_NS = 16
_NW = _NC * _NS
_BPW = _B // _NW
_NBLK = _BPW // 16
_CHUNK = 128
_NCHUNK = _BPW // _CHUNK
_SLOT = 32


def _tc_transpose_body(tt_ref, out_ref):
    y = jnp.concatenate(
        [tt_ref[:, pl.ds(_QB * q, _QB)] for q in range(4)], axis=0)
    z = y.astype(jnp.bfloat16).astype(jnp.float32)
    out_ref[...] = jnp.transpose(z)


@jax.jit
def _tc_transpose(tt):
    return pl.pallas_call(
        _tc_transpose_body,
        grid=(_GRID,),
        in_specs=[pl.BlockSpec((_EMB, _TB), lambda g: (0, g))],
        out_specs=pl.BlockSpec((_QB, 128), lambda g: (g, 0)),
        out_shape=jax.ShapeDtypeStruct((_VP // 4, 128), jnp.float32),
    )(tt)


def _sc_body(idx1_hbm, idx2_hbm, sd_hbm, table_hbm, wb_hbm, out_hbm,
             idx1_v, idx2_v, sd_v, rows1_v, rows2_v, w_v, red_v, o_v, sem):
    wid = lax.axis_index("s") * _NC + lax.axis_index("c")
    base = wid * _BPW
    lane = lax.iota(jnp.int32, 16)
    zeros = jnp.zeros((16,), jnp.float32)

    pltpu.sync_copy(idx1_hbm.at[pl.ds(base, _BPW)], idx1_v)
    pltpu.sync_copy(idx2_hbm.at[pl.ds(base, _BPW)], idx2_v)
    pltpu.sync_copy(sd_hbm.at[pl.ds(base, _BPW)], sd_v)
    pltpu.sync_copy(wb_hbm, w_v)

    for iv in (idx1_v, idx2_v):
        for i in range(_NBLK):
            sl = pl.ds(i * 16, 16)
            t = iv[sl]
            rem = jnp.bitwise_and(t, _TB - 1)
            q = jnp.right_shift(rem, _QB.bit_length() - 1)
            j = jnp.bitwise_and(rem, _QB - 1)
            iv[sl] = (t - rem) + jnp.left_shift(j, 2) + q

    copies = []
    for j in range(_NCHUNK):
        rsl = pl.ds(j * _CHUNK, _CHUNK)
        copies.append(pltpu.async_copy(
            table_hbm.at[idx1_v.at[rsl]], rows1_v.at[rsl], sem))
        copies.append(pltpu.async_copy(
            table_hbm.at[idx2_v.at[rsl]], rows2_v.at[rsl], sem))

    for j in range(16 * _SLOT // 16):
        red_v[pl.ds(16 * j, 16)] = zeros

    w1lo = w_v[pl.ds(0, 16)]
    w1hi = w_v[pl.ds(16, 16)]
    w2lo = w_v[pl.ds(32, 16)]
    w2hi = w_v[pl.ds(48, 16)]
    wtail = w_v[pl.ds(64, 16)]
    w_sd = wtail[0]
    b0 = wtail[1]
    masks = [lane == r for r in range(16)]

    for cp in copies:
        cp.wait()

    def block(i, _):
        sl = pl.ds(i * 16, 16)
        acc = sd_v[sl] * w_sd + b0
        for rr in range(16):
            r = i * 16 + rr
            v = (rows1_v[r, pl.ds(0, 16)] * w1lo
                 + rows1_v[r, pl.ds(16, 16)] * w1hi
                 + rows2_v[r, pl.ds(0, 16)] * w2lo
                 + rows2_v[r, pl.ds(16, 16)] * w2hi)
            s = rr * _SLOT
            red_v[pl.ds(s, 16)] = v
            v = v + red_v[pl.ds(s + 8, 16)]
            red_v[pl.ds(s, 16)] = v
            v = v + red_v[pl.ds(s + 4, 16)]
            red_v[pl.ds(s, 16)] = v
            v = v + red_v[pl.ds(s + 2, 16)]
            red_v[pl.ds(s, 16)] = v
            v = v + red_v[pl.ds(s + 1, 16)]
            acc = acc + jnp.where(masks[rr], v[0], 0.0)
        o_v[sl] = 1.0 / (1.0 + jnp.exp(-acc))
        return _

    lax.fori_loop(0, _NBLK, block, None)

    pltpu.sync_copy(o_v, out_hbm.at[pl.ds(base, _BPW)])


@jax.jit
def _sc_forward(idx1, idx2, sd, table_lin, wb):
    mesh = plsc.VectorSubcoreMesh(core_axis_name="c", subcore_axis_name="s")
    f = functools.partial(
        pl.kernel,
        mesh=mesh,
        compiler_params=pltpu.CompilerParams(use_tc_tiling_on_sc=False),
        out_type=jax.ShapeDtypeStruct((_B,), jnp.float32),
        scratch_types=[
            pltpu.VMEM((_BPW,), jnp.int32),
            pltpu.VMEM((_BPW,), jnp.int32),
            pltpu.VMEM((_BPW,), jnp.float32),
            pltpu.VMEM((_BPW, _EMB), jnp.float32),
            pltpu.VMEM((_BPW, _EMB), jnp.float32),
            pltpu.VMEM((80,), jnp.float32),
            pltpu.VMEM((16 * _SLOT,), jnp.float32),
            pltpu.VMEM((_BPW,), jnp.float32),
            pltpu.SemaphoreType.DMA,
        ],
    )(_sc_body)
    return f(idx1, idx2, sd, table_lin, wb)


def kernel(idsTensor, table, W, b):
    idx1 = idsTensor[:, 0].astype(jnp.int32)
    idx2 = idsTensor[:, 1].astype(jnp.int32)
    sd = idsTensor[:, 2].astype(jnp.bfloat16).astype(jnp.float32)
    wbf = W.reshape(-1).astype(jnp.bfloat16).astype(jnp.float32)
    wb = jnp.concatenate(
        [wbf, b, jnp.zeros((14,), jnp.float32)])
    table_lin = _tc_transpose(table.T).reshape(_VP, _EMB)
    out = _sc_forward(idx1, idx2, sd, table_lin, wb)
    return out.reshape(_B, 1)


# speedup vs baseline: 1.0353x; 1.0088x over previous
import functools

import jax
import jax.numpy as jnp
from jax import lax
from jax.experimental import pallas as pl
from jax.experimental.pallas import tpu as pltpu
from jax.experimental.pallas import tpu_sc as plsc

_EMB = 32
_B = 16384
_V = 1000000
_TB = 65536
_QB = _TB // 4
_GRID = -(-_V // _TB)
_VP = _GRID * _TB
_NC = 2
_NS = 16
_NW = _NC * _NS
_BPW = _B // _NW
_NBLK = _BPW // 16
_CHUNK = 128
_NCHUNK = _BPW // _CHUNK
_SLOT = 32


def _tc_transpose_body(tt_ref, out_ref):
    y = jnp.concatenate(
        [tt_ref[:, pl.ds(_QB * q, _QB)] for q in range(4)], axis=0)
    z = y.astype(jnp.bfloat16).astype(jnp.float32)
    out_ref[...] = jnp.transpose(z)


@jax.jit
def _tc_transpose(tt):
    return pl.pallas_call(
        _tc_transpose_body,
        grid=(_GRID,),
        in_specs=[pl.BlockSpec((_EMB, _TB), lambda g: (0, g))],
        out_specs=pl.BlockSpec((_QB, 128), lambda g: (g, 0)),
        out_shape=jax.ShapeDtypeStruct((_VP // 4, 128), jnp.float32),
    )(tt)


def _sc_body(idx1_hbm, idx2_hbm, sd_hbm, table_hbm, wb_hbm, out_hbm,
             idx1_v, idx2_v, sd_v, rows1_v, rows2_v, w_v, red_v, o_v,
             sem, gsem):
    wid = lax.axis_index("s") * _NC + lax.axis_index("c")
    base = wid * _BPW
    lane = lax.iota(jnp.int32, 16)
    zeros = jnp.zeros((16,), jnp.float32)

    c1 = pltpu.async_copy(idx1_hbm.at[pl.ds(base, _BPW)], idx1_v, sem)
    c2 = pltpu.async_copy(idx2_hbm.at[pl.ds(base, _BPW)], idx2_v, sem)
    c3 = pltpu.async_copy(sd_hbm.at[pl.ds(base, _BPW)], sd_v, sem)
    c4 = pltpu.async_copy(wb_hbm, w_v, sem)
    c1.wait()
    c2.wait()

    copies = []
    bpc = _CHUNK // 16
    for j in range(_NCHUNK):
        for iv in (idx1_v, idx2_v):
            for i in range(j * bpc, (j + 1) * bpc):
                sl = pl.ds(i * 16, 16)
                t = iv[sl]
                rem = jnp.bitwise_and(t, _TB - 1)
                q = jnp.right_shift(rem, _QB.bit_length() - 1)
                jj = jnp.bitwise_and(rem, _QB - 1)
                iv[sl] = (t - rem) + jnp.left_shift(jj, 2) + q
        rsl = pl.ds(j * _CHUNK, _CHUNK)
        copies.append(pltpu.async_copy(
            table_hbm.at[idx1_v.at[rsl]], rows1_v.at[rsl], gsem))
        copies.append(pltpu.async_copy(
            table_hbm.at[idx2_v.at[rsl]], rows2_v.at[rsl], gsem))

    for j in range(16 * _SLOT // 16):
        red_v[pl.ds(16 * j, 16)] = zeros

    c3.wait()
    c4.wait()
    w1lo = w_v[pl.ds(0, 16)]
    w1hi = w_v[pl.ds(16, 16)]
    w2lo = w_v[pl.ds(32, 16)]
    w2hi = w_v[pl.ds(48, 16)]
    wtail = w_v[pl.ds(64, 16)]
    w_sd = wtail[0]
    b0 = wtail[1]
    masks = [lane == r for r in range(16)]

    def block(i, _):
        sl = pl.ds(i * 16, 16)
        acc = sd_v[sl] * w_sd + b0
        for rr in range(16):
            r = i * 16 + rr
            v = (rows1_v[r, pl.ds(0, 16)] * w1lo
                 + rows1_v[r, pl.ds(16, 16)] * w1hi
                 + rows2_v[r, pl.ds(0, 16)] * w2lo
                 + rows2_v[r, pl.ds(16, 16)] * w2hi)
            s = rr * _SLOT
            red_v[pl.ds(s, 16)] = v
            v = v + red_v[pl.ds(s + 8, 16)]
            red_v[pl.ds(s, 16)] = v
            v = v + red_v[pl.ds(s + 4, 16)]
            red_v[pl.ds(s, 16)] = v
            v = v + red_v[pl.ds(s + 2, 16)]
            red_v[pl.ds(s, 16)] = v
            v = v + red_v[pl.ds(s + 1, 16)]
            acc = acc + jnp.where(masks[rr], v[0], 0.0)
        o_v[sl] = 1.0 / (1.0 + jnp.exp(-acc))
        return _

    for j in range(_NCHUNK):
        copies[2 * j].wait()
        copies[2 * j + 1].wait()
        lax.fori_loop(j * bpc, (j + 1) * bpc, block, None)

    pltpu.sync_copy(o_v, out_hbm.at[pl.ds(base, _BPW)])


@jax.jit
def _sc_forward(idx1, idx2, sd, table_lin, wb):
    mesh = plsc.VectorSubcoreMesh(core_axis_name="c", subcore_axis_name="s")
    f = functools.partial(
        pl.kernel,
        mesh=mesh,
        compiler_params=pltpu.CompilerParams(use_tc_tiling_on_sc=False),
        out_type=jax.ShapeDtypeStruct((_B,), jnp.float32),
        scratch_types=[
            pltpu.VMEM((_BPW,), jnp.int32),
            pltpu.VMEM((_BPW,), jnp.int32),
            pltpu.VMEM((_BPW,), jnp.float32),
            pltpu.VMEM((_BPW, _EMB), jnp.float32),
            pltpu.VMEM((_BPW, _EMB), jnp.float32),
            pltpu.VMEM((80,), jnp.float32),
            pltpu.VMEM((16 * _SLOT,), jnp.float32),
            pltpu.VMEM((_BPW,), jnp.float32),
            pltpu.SemaphoreType.DMA,
            pltpu.SemaphoreType.DMA,
        ],
    )(_sc_body)
    return f(idx1, idx2, sd, table_lin, wb)


def kernel(idsTensor, table, W, b):
    idx1 = idsTensor[:, 0].astype(jnp.int32)
    idx2 = idsTensor[:, 1].astype(jnp.int32)
    sd = idsTensor[:, 2].astype(jnp.bfloat16).astype(jnp.float32)
    wbf = W.reshape(-1).astype(jnp.bfloat16).astype(jnp.float32)
    wb = jnp.concatenate(
        [wbf, b, jnp.zeros((14,), jnp.float32)])
    table_lin = _tc_transpose(table.T).reshape(_VP, _EMB)
    out = _sc_forward(idx1, idx2, sd, table_lin, wb)
    return out.reshape(_B, 1)
